# baseline (speedup 1.0000x reference)
import jax
import jax.numpy as jnp
from jax.experimental import pallas as pl
from jax.experimental.pallas import tpu as pltpu


def _fused_body(x_ref, w_ref, adj_ref, out_ref, sup_ref):
    m = pl.program_id(1)

    @pl.when(m == 0)
    def _():
        sup_ref[...] = jnp.dot(x_ref[...], w_ref[...],
                               preferred_element_type=jnp.float32)

    prod = jnp.dot(adj_ref[...], sup_ref[...],
                   preferred_element_type=jnp.float32)
    out_ref[...] = jnp.maximum(prod, 0.0)


@jax.jit
def kernel(input, adj, W):
    n_nodes, in_features = input.shape
    out_features = W.shape[1]

    tm = 512
    ncore = 2
    nm = n_nodes // tm // ncore

    out = pl.pallas_call(
        _fused_body,
        grid=(ncore, nm),
        in_specs=[
            pl.BlockSpec((n_nodes, in_features), lambda c, m: (0, 0)),
            pl.BlockSpec((in_features, out_features), lambda c, m: (0, 0)),
            pl.BlockSpec((tm, n_nodes), lambda c, m, nm=nm: (c * nm + m, 0)),
        ],
        out_specs=pl.BlockSpec(
            (tm, out_features), lambda c, m, nm=nm: (c * nm + m, 0)),
        out_shape=jax.ShapeDtypeStruct((n_nodes, out_features), jnp.float32),
        scratch_shapes=[pltpu.VMEM((n_nodes, out_features), jnp.float32)],
        compiler_params=pltpu.CompilerParams(
            dimension_semantics=("parallel", "arbitrary")),
    )(input, W, adj)
    return out
